# Initial kernel scaffold; baseline (speedup 1.0000x reference)
import jax
import jax.numpy as jnp
from jax.experimental import pallas as pl


def kernel(pair_indices, f_ij, d_ij, atomic_embedding, G, Wi, bi, Wj, bj, Wv, bv, res0_W1, res0_b1, res0_W2, res0_b2, res1_W1, res1_b1, res1_W2, res1_b2, res2_W1, res2_b1, res2_W2, res2_b2):
    raise NotImplementedError("write your pallas kernel here")


# speedup vs baseline: 2.4021x; 2.4021x over previous
import functools

import jax
import jax.numpy as jnp
from jax import lax
from jax.experimental import pallas as pl
from jax.experimental.pallas import tpu as pltpu
from jax.experimental.pallas import tpu_sc as plsc

N_ATOMS = 10000
N_PAIRS = 320000
D = 128
N_RBF = 16
LOG2 = 0.6931471805599453

NC = 2
NS = 16
NW = NC * NS
PPW = N_PAIRS // NW
CHUNK = 80
NCHUNK = PPW // CHUNK
ZROWS = 80
NZCH = N_ATOMS // ZROWS


def _sp(x):
    return jnp.maximum(x, 0.0) + jnp.log1p(jnp.exp(-jnp.abs(x)))


def _pre_body(e_ref, wiT_ref, bi_ref, wjT_ref, bj_ref, xi_ref, y_ref):
    x = _sp(e_ref[...]) - LOG2
    xi_ref[...] = _sp(
        jnp.dot(x, wiT_ref[...], preferred_element_type=jnp.float32) + bi_ref[...])
    y_ref[...] = _sp(
        jnp.dot(x, wjT_ref[...], preferred_element_type=jnp.float32) + bj_ref[...])


def _pre(e, wiT, bi2, wjT, bj2):
    blk = 1000
    grid = N_ATOMS // blk
    return pl.pallas_call(
        _pre_body,
        grid=(grid,),
        in_specs=[
            pl.BlockSpec((blk, D), lambda i: (i, 0)),
            pl.BlockSpec((D, D), lambda i: (0, 0)),
            pl.BlockSpec((1, D), lambda i: (0, 0)),
            pl.BlockSpec((D, D), lambda i: (0, 0)),
            pl.BlockSpec((1, D), lambda i: (0, 0)),
        ],
        out_specs=[
            pl.BlockSpec((blk, D), lambda i: (i, 0)),
            pl.BlockSpec((blk, D), lambda i: (i, 0)),
        ],
        out_shape=[
            jax.ShapeDtypeStruct((N_ATOMS, D), jnp.float32),
            jax.ShapeDtypeStruct((N_ATOMS, D), jnp.float32),
        ],
    )(e, wiT, bi2, wjT, bj2)


def _fp_body(f_ref, gT_ref, fp_ref):
    fp_ref[...] = jnp.dot(f_ref[...], gT_ref[...],
                          preferred_element_type=jnp.float32)


def _fprime(f_ij, gT):
    blk = 8000
    grid = N_PAIRS // blk
    return pl.pallas_call(
        _fp_body,
        grid=(grid,),
        in_specs=[
            pl.BlockSpec((blk, N_RBF), lambda i: (i, 0)),
            pl.BlockSpec((N_RBF, D), lambda i: (0, 0)),
        ],
        out_specs=pl.BlockSpec((blk, D), lambda i: (i, 0)),
        out_shape=jax.ShapeDtypeStruct((N_PAIRS, D), jnp.float32),
    )(f_ij, gT)


def _sc_body(y_hbm, fp_hbm, idxj_hbm, idxi_hbm, out_hbm,
             idxj_v, idxi_v, rows_v, fp_v, zbuf_v, acc_sh, sem):
    c = lax.axis_index("c")
    s = lax.axis_index("s")
    wid = s * NC + c

    zero = jnp.zeros((16,), jnp.float32)

    def _zb(i, carry):
        for v in range(D // 16):
            zbuf_v[i, pl.ds(v * 16, 16)] = zero
        return carry

    lax.fori_loop(0, ZROWS, _zb, 0)

    def _zero(r, carry):
        m = r * NS + s

        @pl.when(m < NZCH)
        def _():
            pltpu.sync_copy(zbuf_v, acc_sh.at[pl.ds(m * ZROWS, ZROWS), :])

        return carry

    lax.fori_loop(0, (NZCH + NS - 1) // NS, _zero, 0)

    plsc.subcore_barrier()

    def _chunk(k, carry):
        base = wid * PPW + k * CHUNK
        pltpu.sync_copy(idxj_hbm.at[pl.ds(base, CHUNK)], idxj_v)
        pltpu.sync_copy(idxi_hbm.at[pl.ds(base, CHUNK)], idxi_v)
        pltpu.sync_copy(fp_hbm.at[pl.ds(base, CHUNK), :], fp_v)
        pltpu.async_copy(y_hbm.at[idxj_v], rows_v, sem).wait()

        def _mul(p, inner):
            for v in range(D // 16):
                sl = pl.ds(v * 16, 16)
                rows_v[p, sl] = rows_v[p, sl] * fp_v[p, sl]
            return inner

        lax.fori_loop(0, CHUNK, _mul, 0)
        pltpu.sync_copy(rows_v, acc_sh.at[idxi_v], add=True)
        return carry

    lax.fori_loop(0, NCHUNK, _chunk, 0)

    plsc.subcore_barrier()

    def _wb(r, carry):
        m = r * NS + s

        @pl.when(m < NZCH)
        def _():
            pltpu.sync_copy(acc_sh.at[pl.ds(m * ZROWS, ZROWS), :], zbuf_v)
            pltpu.sync_copy(zbuf_v, out_hbm.at[c, pl.ds(m * ZROWS, ZROWS), :])

        return carry

    lax.fori_loop(0, (NZCH + NS - 1) // NS, _wb, 0)


def _scatter_sum(y, fp, idx_j, idx_i):
    mesh = plsc.VectorSubcoreMesh(core_axis_name="c", subcore_axis_name="s",
                                  num_cores=NC, num_subcores=NS)
    fn = functools.partial(
        pl.kernel,
        out_type=jax.ShapeDtypeStruct((NC, N_ATOMS, D), jnp.float32),
        mesh=mesh,
        scratch_types=[
            pltpu.VMEM((CHUNK,), jnp.int32),
            pltpu.VMEM((CHUNK,), jnp.int32),
            pltpu.VMEM((CHUNK, D), jnp.float32),
            pltpu.VMEM((CHUNK, D), jnp.float32),
            pltpu.VMEM((ZROWS, D), jnp.float32),
            pltpu.VMEM_SHARED((N_ATOMS, D), jnp.float32),
            pltpu.SemaphoreType.DMA,
        ],
    )(_sc_body)
    return fn(y, fp, idx_j, idx_i)


def _post_body(xi_ref, s0_ref, s1_ref,
               w10_ref, b10_ref, w20_ref, b20_ref,
               w11_ref, b11_ref, w21_ref, b21_ref,
               w12_ref, b12_ref, w22_ref, b22_ref,
               wvT_ref, bv_ref, out_ref):
    v = xi_ref[...] + s0_ref[...] + s1_ref[...]
    for w1, b1, w2, b2 in (
        (w10_ref, b10_ref, w20_ref, b20_ref),
        (w11_ref, b11_ref, w21_ref, b21_ref),
        (w12_ref, b12_ref, w22_ref, b22_ref),
    ):
        h = _sp(jnp.dot(v, w1[...], preferred_element_type=jnp.float32) + b1[...])
        v = v + jnp.dot(h, w2[...], preferred_element_type=jnp.float32) + b2[...]
    out_ref[...] = jnp.dot(_sp(v), wvT_ref[...],
                           preferred_element_type=jnp.float32) + bv_ref[...]


def _post(xi, s0, s1, mats, vecs, wvT, bv2):
    blk = 1000
    grid = N_ATOMS // blk
    full = pl.BlockSpec((D, D), lambda i: (0, 0))
    row = pl.BlockSpec((1, D), lambda i: (0, 0))
    big = pl.BlockSpec((blk, D), lambda i: (i, 0))
    in_specs = [big, big, big]
    args = [xi, s0, s1]
    for m, v in zip(mats, vecs):
        in_specs += [full, row]
        args += [m, v]
    in_specs += [full, row]
    args += [wvT, bv2]
    return pl.pallas_call(
        _post_body,
        grid=(grid,),
        in_specs=in_specs,
        out_specs=big,
        out_shape=jax.ShapeDtypeStruct((N_ATOMS, D), jnp.float32),
    )(*args)


def kernel(pair_indices, f_ij, d_ij, atomic_embedding, G, Wi, bi, Wj, bj, Wv, bv,
           res0_W1, res0_b1, res0_W2, res0_b2,
           res1_W1, res1_b1, res1_W2, res1_b2,
           res2_W1, res2_b1, res2_W2, res2_b2):
    idx_i = pair_indices[0].astype(jnp.int32)
    idx_j = pair_indices[1].astype(jnp.int32)

    xi, y = _pre(atomic_embedding, Wi.T, bi.reshape(1, D), Wj.T, bj.reshape(1, D))
    fp = _fprime(f_ij, G.T)
    acc = _scatter_sum(y, fp, idx_j, idx_i)
    mats = (res0_W1.T, res0_W2.T, res1_W1.T, res1_W2.T, res2_W1.T, res2_W2.T)
    vecs = (res0_b1.reshape(1, D), res0_b2.reshape(1, D),
            res1_b1.reshape(1, D), res1_b2.reshape(1, D),
            res2_b1.reshape(1, D), res2_b2.reshape(1, D))
    return _post(xi, acc[0], acc[1], mats, vecs, Wv.T, bv.reshape(1, D))


# speedup vs baseline: 3.9903x; 1.6612x over previous
import jax
import jax.numpy as jnp
from jax import lax
from jax.experimental import pallas as pl
from jax.experimental.pallas import tpu as pltpu
from jax.experimental.pallas import tpu_sc as plsc

N_ATOMS = 10000
N_PAIRS = 320000
D = 128
N_RBF = 16
LOG2 = 0.6931471805599453

NC = 2
NS = 16
NW = NC * NS
PPW = N_PAIRS // NW
CHUNK = 80
NCHUNK = PPW // CHUNK
SUP = 25
NSUP = NCHUNK // SUP
ZROWS = 80
NZCH = N_ATOMS // ZROWS


def _sp(x):
    return jnp.maximum(x, 0.0) + jnp.log1p(jnp.exp(-jnp.abs(x)))


def _lin_body(e_ref, wT_ref, b_ref, out_ref):
    x = _sp(e_ref[...]) - LOG2
    out_ref[...] = _sp(
        jnp.dot(x, wT_ref[...], preferred_element_type=jnp.float32) + b_ref[...])


def _embed_linear(e, wT, b2):
    blk = 1000
    grid = N_ATOMS // blk
    return pl.pallas_call(
        _lin_body,
        grid=(grid,),
        in_specs=[
            pl.BlockSpec((blk, D), lambda i: (i, 0)),
            pl.BlockSpec((D, D), lambda i: (0, 0)),
            pl.BlockSpec((1, D), lambda i: (0, 0)),
        ],
        out_specs=pl.BlockSpec((blk, D), lambda i: (i, 0)),
        out_shape=jax.ShapeDtypeStruct((N_ATOMS, D), jnp.float32),
    )(e, wT, b2)


def _fp_body(f_ref, gT_ref, fp_ref):
    fp_ref[...] = jnp.dot(f_ref[...], gT_ref[...],
                          preferred_element_type=jnp.float32)


def _fprime(f_ij, gT):
    blk = 8000
    grid = N_PAIRS // blk
    return pl.pallas_call(
        _fp_body,
        grid=(grid,),
        in_specs=[
            pl.BlockSpec((blk, N_RBF), lambda i: (i, 0)),
            pl.BlockSpec((N_RBF, D), lambda i: (0, 0)),
        ],
        out_specs=pl.BlockSpec((blk, D), lambda i: (i, 0)),
        out_shape=jax.ShapeDtypeStruct((N_PAIRS, D), jnp.float32),
    )(f_ij, gT)


def _sc_body(y_hbm, fp_hbm, idxc_hbm, out_hbm,
             idxb, rows0, rows1, fpb0, fpb1, acc_sh,
             semg0, semg1, semf0, semf1):
    c = lax.axis_index("c")
    s = lax.axis_index("s")
    wid = s * NC + c
    bufs = ((rows0, fpb0, semg0, semf0), (rows1, fpb1, semg1, semf1))

    zero = jnp.zeros((16,), jnp.float32)

    def _zb(i, carry):
        for v in range(D // 16):
            fpb0[i, pl.ds(v * 16, 16)] = zero
        return carry

    lax.fori_loop(0, ZROWS, _zb, 0)

    def _zero(r, carry):
        m = r * NS + s

        @pl.when(m < NZCH)
        def _():
            pltpu.sync_copy(fpb0, acc_sh.at[pl.ds(m * ZROWS, ZROWS), :])

        return carry

    lax.fori_loop(0, (NZCH + NS - 1) // NS, _zero, 0)

    plsc.subcore_barrier()

    def _issue(k, lk, idxb, rows, fpb, semg, semf):
        pltpu.async_copy(y_hbm.at[idxb.at[lk, 0]], rows, semg)
        pltpu.async_copy(fp_hbm.at[pl.ds(wid * PPW + k * CHUNK, CHUNK), :],
                         fpb, semf)

    def _process(k, lk, idxb, rows, fpb, semg, semf):
        pltpu.make_async_copy(y_hbm.at[idxb.at[lk, 0]], rows, semg).wait()
        pltpu.make_async_copy(fp_hbm.at[pl.ds(wid * PPW + k * CHUNK, CHUNK), :],
                              fpb, semf).wait()

        @plsc.parallel_loop(0, CHUNK, unroll=4)
        def _mul(p):
            for v in range(D // 16):
                sl = pl.ds(v * 16, 16)
                rows[p, sl] = rows[p, sl] * fpb[p, sl]

        pltpu.sync_copy(rows, acc_sh.at[idxb.at[lk, 1]], add=True)

    def _super(su, carry):
        kb = su * SUP
        pltpu.sync_copy(idxc_hbm.at[wid, su], idxb)
        _issue(kb, 0, idxb, *bufs[0])

        def _pair(g, inner):
            k0 = kb + 2 * g
            l0 = 2 * g
            _issue(k0 + 1, l0 + 1, idxb, *bufs[1])
            _process(k0, l0, idxb, *bufs[0])
            _issue(k0 + 2, l0 + 2, idxb, *bufs[0])
            _process(k0 + 1, l0 + 1, idxb, *bufs[1])
            return inner

        lax.fori_loop(0, SUP // 2, _pair, 0)
        _process(kb + SUP - 1, SUP - 1, idxb, *bufs[0])
        return carry

    lax.fori_loop(0, NSUP, _super, 0)

    plsc.subcore_barrier()

    def _wb(r, carry):
        m = r * NS + s

        @pl.when(m < NZCH)
        def _():
            pltpu.sync_copy(acc_sh.at[pl.ds(m * ZROWS, ZROWS), :], fpb0)
            pltpu.sync_copy(fpb0, out_hbm.at[c, pl.ds(m * ZROWS, ZROWS), :])

        return carry

    lax.fori_loop(0, (NZCH + NS - 1) // NS, _wb, 0)


def _scatter_sum(y, fp, idxc):
    mesh = plsc.VectorSubcoreMesh(core_axis_name="c", subcore_axis_name="s",
                                  num_cores=NC, num_subcores=NS)
    fn = pl.kernel(
        _sc_body,
        out_type=jax.ShapeDtypeStruct((NC, N_ATOMS, D), jnp.float32),
        mesh=mesh,
        scratch_types=[
            pltpu.VMEM((SUP, 2, CHUNK), jnp.int32),
            pltpu.VMEM((CHUNK, D), jnp.float32),
            pltpu.VMEM((CHUNK, D), jnp.float32),
            pltpu.VMEM((CHUNK, D), jnp.float32),
            pltpu.VMEM((CHUNK, D), jnp.float32),
            pltpu.VMEM_SHARED((N_ATOMS, D), jnp.float32),
            pltpu.SemaphoreType.DMA,
            pltpu.SemaphoreType.DMA,
            pltpu.SemaphoreType.DMA,
            pltpu.SemaphoreType.DMA,
        ],
    )
    return fn(y, fp, idxc)


def _post_body(xi_ref, s0_ref, s1_ref,
               w10_ref, b10_ref, w20_ref, b20_ref,
               w11_ref, b11_ref, w21_ref, b21_ref,
               w12_ref, b12_ref, w22_ref, b22_ref,
               wvT_ref, bv_ref, out_ref):
    v = xi_ref[...] + s0_ref[...] + s1_ref[...]
    for w1, b1, w2, b2 in (
        (w10_ref, b10_ref, w20_ref, b20_ref),
        (w11_ref, b11_ref, w21_ref, b21_ref),
        (w12_ref, b12_ref, w22_ref, b22_ref),
    ):
        h = _sp(jnp.dot(v, w1[...], preferred_element_type=jnp.float32) + b1[...])
        v = v + jnp.dot(h, w2[...], preferred_element_type=jnp.float32) + b2[...]
    out_ref[...] = jnp.dot(_sp(v), wvT_ref[...],
                           preferred_element_type=jnp.float32) + bv_ref[...]


def _post(xi, s0, s1, mats, vecs, wvT, bv2):
    blk = 1000
    grid = N_ATOMS // blk
    full = pl.BlockSpec((D, D), lambda i: (0, 0))
    row = pl.BlockSpec((1, D), lambda i: (0, 0))
    big = pl.BlockSpec((blk, D), lambda i: (i, 0))
    in_specs = [big, big, big]
    args = [xi, s0, s1]
    for m, v in zip(mats, vecs):
        in_specs += [full, row]
        args += [m, v]
    in_specs += [full, row]
    args += [wvT, bv2]
    return pl.pallas_call(
        _post_body,
        grid=(grid,),
        in_specs=in_specs,
        out_specs=big,
        out_shape=jax.ShapeDtypeStruct((N_ATOMS, D), jnp.float32),
    )(*args)


def kernel(pair_indices, f_ij, d_ij, atomic_embedding, G, Wi, bi, Wj, bj, Wv, bv,
           res0_W1, res0_b1, res0_W2, res0_b2,
           res1_W1, res1_b1, res1_W2, res1_b2,
           res2_W1, res2_b1, res2_W2, res2_b2):
    idx = pair_indices.astype(jnp.int32)
    idxc = jnp.stack(
        [idx[1].reshape(NW, NCHUNK, CHUNK), idx[0].reshape(NW, NCHUNK, CHUNK)],
        axis=2).reshape(NW, NSUP, SUP, 2, CHUNK)

    y = _embed_linear(atomic_embedding, Wj.T, bj.reshape(1, D))
    fp = _fprime(f_ij, G.T)
    acc = _scatter_sum(y, fp, idxc)
    xi = _embed_linear(atomic_embedding, Wi.T, bi.reshape(1, D))
    mats = (res0_W1.T, res0_W2.T, res1_W1.T, res1_W2.T, res2_W1.T, res2_W2.T)
    vecs = (res0_b1.reshape(1, D), res0_b2.reshape(1, D),
            res1_b1.reshape(1, D), res1_b2.reshape(1, D),
            res2_b1.reshape(1, D), res2_b2.reshape(1, D))
    return _post(xi, acc[0], acc[1], mats, vecs, Wv.T, bv.reshape(1, D))


# speedup vs baseline: 5.4399x; 1.3633x over previous
import jax
import jax.numpy as jnp
from jax import lax
from jax.experimental import pallas as pl
from jax.experimental.pallas import tpu as pltpu
from jax.experimental.pallas import tpu_sc as plsc

N_ATOMS = 10000
N_PAIRS = 320000
D = 128
N_RBF = 16
LOG2 = 0.6931471805599453

NC = 2
NS = 16
NW = NC * NS
PPW = N_PAIRS // NW
CHUNK = 80
NCHUNK = PPW // CHUNK
SUP = 25
NSUP = NCHUNK // SUP
ZROWS = 80
NZCH = N_ATOMS // ZROWS


def _sp(x):
    return jnp.maximum(x, 0.0) + jnp.log1p(jnp.exp(-jnp.abs(x)))


def _lin_body(e_ref, wT_ref, b_ref, out_ref):
    x = _sp(e_ref[...]) - LOG2
    out_ref[...] = _sp(
        jnp.dot(x, wT_ref[...], preferred_element_type=jnp.float32) + b_ref[...])


def _embed_linear(e, wT, b2):
    blk = 1000
    grid = N_ATOMS // blk
    return pl.pallas_call(
        _lin_body,
        grid=(grid,),
        in_specs=[
            pl.BlockSpec((blk, D), lambda i: (i, 0)),
            pl.BlockSpec((D, D), lambda i: (0, 0)),
            pl.BlockSpec((1, D), lambda i: (0, 0)),
        ],
        out_specs=pl.BlockSpec((blk, D), lambda i: (i, 0)),
        out_shape=jax.ShapeDtypeStruct((N_ATOMS, D), jnp.float32),
    )(e, wT, b2)


def _fp_body(fT_ref, gT_ref, fp_ref):
    fp_ref[...] = lax.dot_general(
        fT_ref[...], gT_ref[...],
        dimension_numbers=(((0,), (0,)), ((), ())),
        preferred_element_type=jnp.float32)


def _fprime(fT, gT):
    blk = 16000
    grid = N_PAIRS // blk
    return pl.pallas_call(
        _fp_body,
        grid=(grid,),
        in_specs=[
            pl.BlockSpec((N_RBF, blk), lambda i: (0, i)),
            pl.BlockSpec((N_RBF, D), lambda i: (0, 0)),
        ],
        out_specs=pl.BlockSpec((blk, D), lambda i: (i, 0)),
        out_shape=jax.ShapeDtypeStruct((N_PAIRS, D), jnp.float32),
    )(fT, gT)


def _sc_body(y_hbm, fp_hbm, idxc_hbm, out_hbm,
             idxb, rows0, rows1, fpb0, fpb1, acc_sh,
             semg0, semg1, semf0, semf1, sems0, sems1):
    c = lax.axis_index("c")
    s = lax.axis_index("s")
    wid = s * NC + c
    bufs = ((rows0, fpb0, semg0, semf0, sems0),
            (rows1, fpb1, semg1, semf1, sems1))

    zero = jnp.zeros((16,), jnp.float32)

    def _zb(i, carry):
        for v in range(D // 16):
            fpb0[i, pl.ds(v * 16, 16)] = zero
        return carry

    lax.fori_loop(0, ZROWS, _zb, 0)

    def _zero(r, carry):
        m = r * NS + s

        @pl.when(m < NZCH)
        def _():
            pltpu.sync_copy(fpb0, acc_sh.at[pl.ds(m * ZROWS, ZROWS), :])

        return carry

    lax.fori_loop(0, (NZCH + NS - 1) // NS, _zero, 0)

    plsc.subcore_barrier()

    def _issue(k, lk, idxb, rows, fpb, semg, semf, sems, drain=True):
        if drain:
            pltpu.make_async_copy(rows, acc_sh.at[idxb.at[lk, 1]], sems).wait()
        pltpu.async_copy(y_hbm.at[idxb.at[lk, 0]], rows, semg)
        pltpu.async_copy(fp_hbm.at[pl.ds(wid * PPW + k * CHUNK, CHUNK), :],
                         fpb, semf)

    def _process(k, lk, idxb, rows, fpb, semg, semf, sems):
        pltpu.make_async_copy(y_hbm.at[idxb.at[lk, 0]], rows, semg).wait()
        pltpu.make_async_copy(fp_hbm.at[pl.ds(wid * PPW + k * CHUNK, CHUNK), :],
                              fpb, semf).wait()

        @plsc.parallel_loop(0, CHUNK, unroll=4)
        def _mul(p):
            for v in range(D // 16):
                sl = pl.ds(v * 16, 16)
                rows[p, sl] = rows[p, sl] * fpb[p, sl]

        pltpu.async_copy(rows, acc_sh.at[idxb.at[lk, 1]], sems, add=True)

    def _super(su, first):
        kb = su * SUP
        pltpu.sync_copy(idxc_hbm.at[wid, su], idxb)
        _issue(kb, 0, idxb, *bufs[0], drain=not first)
        if first:
            _issue(kb + 1, 1, idxb, *bufs[1], drain=False)
            _process(kb, 0, idxb, *bufs[0])
            _issue(kb + 2, 2, idxb, *bufs[0])
            _process(kb + 1, 1, idxb, *bufs[1])

        def _pair(g, inner):
            k0 = kb + 2 * g
            l0 = 2 * g
            _issue(k0 + 1, l0 + 1, idxb, *bufs[1])
            _process(k0, l0, idxb, *bufs[0])
            _issue(k0 + 2, l0 + 2, idxb, *bufs[0])
            _process(k0 + 1, l0 + 1, idxb, *bufs[1])
            return inner

        lax.fori_loop(1 if first else 0, SUP // 2, _pair, 0)
        _process(kb + SUP - 1, SUP - 1, idxb, *bufs[0])

    _super(0, True)

    def _super_rest(su, carry):
        _super(su, False)
        return carry

    lax.fori_loop(1, NSUP, _super_rest, 0)

    pltpu.make_async_copy(rows0, acc_sh.at[idxb.at[SUP - 1, 1]], sems0).wait()
    pltpu.make_async_copy(rows1, acc_sh.at[idxb.at[SUP - 2, 1]], sems1).wait()

    plsc.subcore_barrier()

    def _wb(r, carry):
        m = r * NS + s

        @pl.when(m < NZCH)
        def _():
            pltpu.sync_copy(acc_sh.at[pl.ds(m * ZROWS, ZROWS), :], fpb0)
            pltpu.sync_copy(fpb0, out_hbm.at[c, pl.ds(m * ZROWS, ZROWS), :])

        return carry

    lax.fori_loop(0, (NZCH + NS - 1) // NS, _wb, 0)


def _scatter_sum(y, fp, idxc):
    mesh = plsc.VectorSubcoreMesh(core_axis_name="c", subcore_axis_name="s",
                                  num_cores=NC, num_subcores=NS)
    fn = pl.kernel(
        _sc_body,
        out_type=jax.ShapeDtypeStruct((NC, N_ATOMS, D), jnp.float32),
        mesh=mesh,
        scratch_types=[
            pltpu.VMEM((SUP, 2, CHUNK), jnp.int32),
            pltpu.VMEM((CHUNK, D), jnp.float32),
            pltpu.VMEM((CHUNK, D), jnp.float32),
            pltpu.VMEM((CHUNK, D), jnp.float32),
            pltpu.VMEM((CHUNK, D), jnp.float32),
            pltpu.VMEM_SHARED((N_ATOMS, D), jnp.float32),
            pltpu.SemaphoreType.DMA,
            pltpu.SemaphoreType.DMA,
            pltpu.SemaphoreType.DMA,
            pltpu.SemaphoreType.DMA,
            pltpu.SemaphoreType.DMA,
            pltpu.SemaphoreType.DMA,
        ],
    )
    return fn(y, fp, idxc)


def _post_body(xi_ref, s0_ref, s1_ref,
               w10_ref, b10_ref, w20_ref, b20_ref,
               w11_ref, b11_ref, w21_ref, b21_ref,
               w12_ref, b12_ref, w22_ref, b22_ref,
               wvT_ref, bv_ref, out_ref):
    v = xi_ref[...] + s0_ref[...] + s1_ref[...]
    for w1, b1, w2, b2 in (
        (w10_ref, b10_ref, w20_ref, b20_ref),
        (w11_ref, b11_ref, w21_ref, b21_ref),
        (w12_ref, b12_ref, w22_ref, b22_ref),
    ):
        h = _sp(jnp.dot(v, w1[...], preferred_element_type=jnp.float32) + b1[...])
        v = v + jnp.dot(h, w2[...], preferred_element_type=jnp.float32) + b2[...]
    out_ref[...] = jnp.dot(_sp(v), wvT_ref[...],
                           preferred_element_type=jnp.float32) + bv_ref[...]


def _post(xi, s0, s1, mats, vecs, wvT, bv2):
    blk = 1000
    grid = N_ATOMS // blk
    full = pl.BlockSpec((D, D), lambda i: (0, 0))
    row = pl.BlockSpec((1, D), lambda i: (0, 0))
    big = pl.BlockSpec((blk, D), lambda i: (i, 0))
    in_specs = [big, big, big]
    args = [xi, s0, s1]
    for m, v in zip(mats, vecs):
        in_specs += [full, row]
        args += [m, v]
    in_specs += [full, row]
    args += [wvT, bv2]
    return pl.pallas_call(
        _post_body,
        grid=(grid,),
        in_specs=in_specs,
        out_specs=big,
        out_shape=jax.ShapeDtypeStruct((N_ATOMS, D), jnp.float32),
    )(*args)


def kernel(pair_indices, f_ij, d_ij, atomic_embedding, G, Wi, bi, Wj, bj, Wv, bv,
           res0_W1, res0_b1, res0_W2, res0_b2,
           res1_W1, res1_b1, res1_W2, res1_b2,
           res2_W1, res2_b1, res2_W2, res2_b2):
    idx = pair_indices.astype(jnp.int32)
    idxc = jnp.stack(
        [idx[1].reshape(NW, NCHUNK, CHUNK), idx[0].reshape(NW, NCHUNK, CHUNK)],
        axis=2).reshape(NW, NSUP, SUP, 2, CHUNK)

    y = _embed_linear(atomic_embedding, Wj.T, bj.reshape(1, D))
    fp = _fprime(f_ij.T, G.T)
    acc = _scatter_sum(y, fp, idxc)
    xi = _embed_linear(atomic_embedding, Wi.T, bi.reshape(1, D))
    mats = (res0_W1.T, res0_W2.T, res1_W1.T, res1_W2.T, res2_W1.T, res2_W2.T)
    vecs = (res0_b1.reshape(1, D), res0_b2.reshape(1, D),
            res1_b1.reshape(1, D), res1_b2.reshape(1, D),
            res2_b1.reshape(1, D), res2_b2.reshape(1, D))
    return _post(xi, acc[0], acc[1], mats, vecs, Wv.T, bv.reshape(1, D))


# speedup vs baseline: 5.7544x; 1.0578x over previous
import jax
import jax.numpy as jnp
import numpy as np
from jax import lax
from jax.experimental import pallas as pl
from jax.experimental.pallas import tpu as pltpu
from jax.experimental.pallas import tpu_sc as plsc

N_ATOMS = 10000
N_PAIRS = 320000
D = 128
N_RBF = 16
LOG2 = 0.6931471805599453

NC = 2
NS = 16
NW = NC * NS
PPW = N_PAIRS // NW
CHUNK = 80
NCHUNK = PPW // CHUNK
SUP = 25
NSUP = NCHUNK // SUP
ZROWS = 80
NZCH = N_ATOMS // ZROWS


def _sp(x):
    return jnp.maximum(x, 0.0) + jnp.log1p(jnp.exp(-jnp.abs(x)))


def _lin_body(e_ref, wT_ref, b_ref, out_ref):
    x = _sp(e_ref[...]) - LOG2
    out_ref[...] = _sp(
        jnp.dot(x, wT_ref[...], preferred_element_type=jnp.float32) + b_ref[...])


def _embed_linear(e, wT, b2):
    blk = 1000
    grid = N_ATOMS // blk
    return pl.pallas_call(
        _lin_body,
        grid=(grid,),
        in_specs=[
            pl.BlockSpec((blk, D), lambda i: (i, 0)),
            pl.BlockSpec((D, D), lambda i: (0, 0)),
            pl.BlockSpec((1, D), lambda i: (0, 0)),
        ],
        out_specs=pl.BlockSpec((blk, D), lambda i: (i, 0)),
        out_shape=jax.ShapeDtypeStruct((N_ATOMS, D), jnp.float32),
    )(e, wT, b2)


def _fp_body(fT_ref, gT_ref, fp_ref):
    fp_ref[...] = lax.dot_general(fT_ref[...], gT_ref[...],
                                  dimension_numbers=(((0,), (0,)), ((), ())),
                                  preferred_element_type=jnp.float32)


def _fprime(fT, gT):
    blk = 16000
    grid = N_PAIRS // blk
    return pl.pallas_call(
        _fp_body,
        grid=(grid,),
        in_specs=[
            pl.BlockSpec((N_RBF, blk), lambda i: (0, i)),
            pl.BlockSpec((N_RBF, D), lambda i: (0, 0)),
        ],
        out_specs=pl.BlockSpec((blk, D), lambda i: (i, 0)),
        out_shape=jax.ShapeDtypeStruct((N_PAIRS, D), jnp.float32),
    )(fT, gT)


def _sc_body(y_hbm, fp_hbm, idxc_hbm, out_hbm,
             idxbj, idxbi, rows0, rows1, fpb0, fpb1, acc_sh,
             semg0, semg1, semf0, semf1, sems0, sems1):
    c = lax.axis_index("c")
    s = lax.axis_index("s")
    wid = s * NC + c
    bufs = ((rows0, fpb0, semg0, semf0, sems0),
            (rows1, fpb1, semg1, semf1, sems1))

    zero = jnp.zeros((16,), jnp.float32)

    def _zb(i, carry):
        for v in range(D // 16):
            rows0[i, pl.ds(v * 16, 16)] = zero
        return carry

    lax.fori_loop(0, ZROWS, _zb, 0)

    def _zero(r, carry):
        m = r * NS + s

        @pl.when(m < NZCH)
        def _():
            pltpu.sync_copy(rows0, acc_sh.at[pl.ds(m * ZROWS, ZROWS), :])

        return carry

    lax.fori_loop(0, (NZCH + NS - 1) // NS, _zero, 0)

    plsc.subcore_barrier()

    def _issue(k, lk, _unused, rows, fpb, semg, semf, sems, drain=True):
        if drain:
            pltpu.make_async_copy(rows, acc_sh.at[idxbi.at[lk]], sems).wait()
        pltpu.async_copy(y_hbm.at[idxbj.at[lk]], rows, semg)
        pltpu.async_copy(fp_hbm.at[pl.ds(wid * PPW + k * CHUNK, CHUNK), :],
                         fpb, semf)

    def _process(k, lk, _unused, rows, fpb, semg, semf, sems):
        pltpu.make_async_copy(y_hbm.at[idxbj.at[lk]], rows, semg).wait()
        pltpu.make_async_copy(fp_hbm.at[pl.ds(wid * PPW + k * CHUNK, CHUNK), :],
                              fpb, semf).wait()

        @plsc.parallel_loop(0, CHUNK, unroll=4)
        def _mul(p):
            for v in range(D // 16):
                sl = pl.ds(v * 16, 16)
                rows[p, sl] = rows[p, sl] * fpb[p, sl]

        pltpu.async_copy(rows, acc_sh.at[idxbi.at[lk]], sems, add=True)

    def _super(su, first):
        kb = su * SUP
        pltpu.sync_copy(idxc_hbm.at[1, wid, su], idxbj)
        pltpu.sync_copy(idxc_hbm.at[0, wid, su], idxbi)
        _issue(kb, 0, None, *bufs[0], drain=not first)
        if first:
            _issue(kb + 1, 1, None, *bufs[1], drain=False)
            _process(kb, 0, None, *bufs[0])
            _issue(kb + 2, 2, None, *bufs[0])
            _process(kb + 1, 1, None, *bufs[1])

        def _pair(g, inner):
            k0 = kb + 2 * g
            l0 = 2 * g
            _issue(k0 + 1, l0 + 1, None, *bufs[1])
            _process(k0, l0, None, *bufs[0])
            _issue(k0 + 2, l0 + 2, None, *bufs[0])
            _process(k0 + 1, l0 + 1, None, *bufs[1])
            return inner

        lax.fori_loop(1 if first else 0, SUP // 2, _pair, 0)
        _process(kb + SUP - 1, SUP - 1, None, *bufs[0])

    _super(0, True)

    def _super_rest(su, carry):
        _super(su, False)
        return carry

    lax.fori_loop(1, NSUP, _super_rest, 0)

    pltpu.make_async_copy(rows0, acc_sh.at[idxbi.at[SUP - 1]], sems0).wait()
    pltpu.make_async_copy(rows1, acc_sh.at[idxbi.at[SUP - 2]], sems1).wait()

    plsc.subcore_barrier()

    def _wb(r, carry):
        m = r * NS + s

        @pl.when(m < NZCH)
        def _():
            pltpu.sync_copy(acc_sh.at[pl.ds(m * ZROWS, ZROWS), :], rows0)
            pltpu.sync_copy(rows0, out_hbm.at[c, pl.ds(m * ZROWS, ZROWS), :])

        return carry

    lax.fori_loop(0, (NZCH + NS - 1) // NS, _wb, 0)


def _scatter_sum(y, fp, idxc):
    mesh = plsc.VectorSubcoreMesh(core_axis_name="c", subcore_axis_name="s",
                                  num_cores=NC, num_subcores=NS)
    fn = pl.kernel(
        _sc_body,
        out_type=jax.ShapeDtypeStruct((NC, N_ATOMS, D), jnp.float32),
        mesh=mesh,
        scratch_types=[
            pltpu.VMEM((SUP, CHUNK), jnp.int32),
            pltpu.VMEM((SUP, CHUNK), jnp.int32),
            pltpu.VMEM((CHUNK, D), jnp.float32),
            pltpu.VMEM((CHUNK, D), jnp.float32),
            pltpu.VMEM((CHUNK, D), jnp.float32),
            pltpu.VMEM((CHUNK, D), jnp.float32),
            pltpu.VMEM_SHARED((N_ATOMS, D), jnp.float32),
            pltpu.SemaphoreType.DMA,
            pltpu.SemaphoreType.DMA,
            pltpu.SemaphoreType.DMA,
            pltpu.SemaphoreType.DMA,
            pltpu.SemaphoreType.DMA,
            pltpu.SemaphoreType.DMA,
        ],
    )
    return fn(y, fp, idxc)


def _post_body(xi_ref, s0_ref, s1_ref,
               w10_ref, b10_ref, w20_ref, b20_ref,
               w11_ref, b11_ref, w21_ref, b21_ref,
               w12_ref, b12_ref, w22_ref, b22_ref,
               wvT_ref, bv_ref, out_ref):
    v = xi_ref[...] + s0_ref[...] + s1_ref[...]
    for w1, b1, w2, b2 in (
        (w10_ref, b10_ref, w20_ref, b20_ref),
        (w11_ref, b11_ref, w21_ref, b21_ref),
        (w12_ref, b12_ref, w22_ref, b22_ref),
    ):
        h = _sp(jnp.dot(v, w1[...], preferred_element_type=jnp.float32) + b1[...])
        v = v + jnp.dot(h, w2[...], preferred_element_type=jnp.float32) + b2[...]
    out_ref[...] = jnp.dot(_sp(v), wvT_ref[...],
                           preferred_element_type=jnp.float32) + bv_ref[...]


def _post(xi, s0, s1, mats, vecs, wvT, bv2):
    blk = 1000
    grid = N_ATOMS // blk
    full = pl.BlockSpec((D, D), lambda i: (0, 0))
    row = pl.BlockSpec((1, D), lambda i: (0, 0))
    big = pl.BlockSpec((blk, D), lambda i: (i, 0))
    in_specs = [big, big, big]
    args = [xi, s0, s1]
    for m, v in zip(mats, vecs):
        in_specs += [full, row]
        args += [m, v]
    in_specs += [full, row]
    args += [wvT, bv2]
    return pl.pallas_call(
        _post_body,
        grid=(grid,),
        in_specs=in_specs,
        out_specs=big,
        out_shape=jax.ShapeDtypeStruct((N_ATOMS, D), jnp.float32),
    )(*args)


def kernel(pair_indices, f_ij, d_ij, atomic_embedding, G, Wi, bi, Wj, bj, Wv, bv,
           res0_W1, res0_b1, res0_W2, res0_b2,
           res1_W1, res1_b1, res1_W2, res1_b2,
           res2_W1, res2_b1, res2_W2, res2_b2):
    idxc = pair_indices.astype(jnp.int32).reshape(2, NW, NSUP, SUP, CHUNK)

    y = _embed_linear(atomic_embedding, Wj.T, bj.reshape(1, D))
    fp = _fprime(f_ij.T, G.T)
    acc = _scatter_sum(y, fp, idxc)
    xi = _embed_linear(atomic_embedding, Wi.T, bi.reshape(1, D))
    mats = (res0_W1.T, res0_W2.T, res1_W1.T, res1_W2.T, res2_W1.T, res2_W2.T)
    vecs = (res0_b1.reshape(1, D), res0_b2.reshape(1, D),
            res1_b1.reshape(1, D), res1_b2.reshape(1, D),
            res2_b1.reshape(1, D), res2_b2.reshape(1, D))
    return _post(xi, acc[0], acc[1], mats, vecs, Wv.T, bv.reshape(1, D))


# speedup vs baseline: 5.7774x; 1.0040x over previous
import jax
import jax.numpy as jnp
import numpy as np
from jax import lax
from jax.experimental import pallas as pl
from jax.experimental.pallas import tpu as pltpu
from jax.experimental.pallas import tpu_sc as plsc

N_ATOMS = 10000
N_PAIRS = 320000
D = 128
N_RBF = 16
LOG2 = 0.6931471805599453

NC = 2
NS = 16
NW = NC * NS
PPW = N_PAIRS // NW
CHUNK = 80
NCHUNK = PPW // CHUNK
SUP = 25
NSUP = NCHUNK // SUP
ZROWS = 80
NZCH = N_ATOMS // ZROWS


def _sp(x):
    return jnp.maximum(x, 0.0) + jnp.log1p(jnp.exp(-jnp.abs(x)))


def _lin_body(e_ref, wT_ref, b_ref, out_ref):
    x = _sp(e_ref[...]) - LOG2
    out_ref[...] = _sp(
        jnp.dot(x, wT_ref[...], preferred_element_type=jnp.float32) + b_ref[...])


def _embed_linear(e, wT, b2):
    blk = 1000
    grid = N_ATOMS // blk
    return pl.pallas_call(
        _lin_body,
        grid=(grid,),
        in_specs=[
            pl.BlockSpec((blk, D), lambda i: (i, 0)),
            pl.BlockSpec((D, D), lambda i: (0, 0)),
            pl.BlockSpec((1, D), lambda i: (0, 0)),
        ],
        out_specs=pl.BlockSpec((blk, D), lambda i: (i, 0)),
        out_shape=jax.ShapeDtypeStruct((N_ATOMS, D), jnp.float32),
    )(e, wT, b2)


def _fp_body(fT_ref, gT_ref, fp_ref):
    fp_ref[...] = lax.dot_general(fT_ref[...], gT_ref[...],
                                  dimension_numbers=(((0,), (0,)), ((), ())),
                                  preferred_element_type=jnp.float32)


def _fprime(fT, gT):
    blk = 16000
    grid = N_PAIRS // blk
    return pl.pallas_call(
        _fp_body,
        grid=(grid,),
        in_specs=[
            pl.BlockSpec((N_RBF, blk), lambda i: (0, i)),
            pl.BlockSpec((N_RBF, D), lambda i: (0, 0)),
        ],
        out_specs=pl.BlockSpec((blk, D), lambda i: (i, 0)),
        out_shape=jax.ShapeDtypeStruct((N_PAIRS, D), jnp.float32),
    )(fT, gT)


def _sc_body(y_hbm, fp_hbm, idxc_hbm, out_hbm,
             idxbj, idxbi, rows0, rows1, fpb0, fpb1, acc_sh,
             semg0, semg1, semf0, semf1, sems0, sems1):
    c = lax.axis_index("c")
    s = lax.axis_index("s")
    wid = s * NC + c
    bufs = ((rows0, fpb0, semg0, semf0, sems0),
            (rows1, fpb1, semg1, semf1, sems1))

    zero = jnp.zeros((16,), jnp.float32)

    def _zb(i, carry):
        for v in range(D // 16):
            rows0[i, pl.ds(v * 16, 16)] = zero
        return carry

    lax.fori_loop(0, ZROWS, _zb, 0)

    def _zero(r, carry):
        m = r * NS + s

        @pl.when(m < NZCH)
        def _():
            pltpu.sync_copy(rows0, acc_sh.at[pl.ds(m * ZROWS, ZROWS), :])

        return carry

    lax.fori_loop(0, (NZCH + NS - 1) // NS, _zero, 0)

    plsc.subcore_barrier()

    def _issue(k, lk, _unused, rows, fpb, semg, semf, sems, drain=True):
        if drain:
            pltpu.make_async_copy(rows, acc_sh.at[idxbi.at[lk]], sems).wait()
        pltpu.async_copy(y_hbm.at[idxbj.at[lk]], rows, semg)
        pltpu.async_copy(fp_hbm.at[pl.ds(wid * PPW + k * CHUNK, CHUNK), :],
                         fpb, semf)

    def _process(k, lk, _unused, rows, fpb, semg, semf, sems):
        pltpu.make_async_copy(y_hbm.at[idxbj.at[lk]], rows, semg).wait()
        pltpu.make_async_copy(fp_hbm.at[pl.ds(wid * PPW + k * CHUNK, CHUNK), :],
                              fpb, semf).wait()

        @plsc.parallel_loop(0, CHUNK, unroll=4)
        def _mul(p):
            for v in range(D // 16):
                sl = pl.ds(v * 16, 16)
                rows[p, sl] = rows[p, sl] * fpb[p, sl]

        pltpu.async_copy(rows, acc_sh.at[idxbi.at[lk]], sems, add=True)

    def _super(su, first):
        kb = su * SUP
        pltpu.sync_copy(idxc_hbm.at[1, wid, su], idxbj)
        pltpu.sync_copy(idxc_hbm.at[0, wid, su], idxbi)
        _issue(kb, 0, None, *bufs[0], drain=not first)
        if first:
            _issue(kb + 1, 1, None, *bufs[1], drain=False)
            _process(kb, 0, None, *bufs[0])
            _issue(kb + 2, 2, None, *bufs[0])
            _process(kb + 1, 1, None, *bufs[1])

        def _pair(g, inner):
            k0 = kb + 2 * g
            l0 = 2 * g
            _issue(k0 + 1, l0 + 1, None, *bufs[1])
            _process(k0, l0, None, *bufs[0])
            _issue(k0 + 2, l0 + 2, None, *bufs[0])
            _process(k0 + 1, l0 + 1, None, *bufs[1])
            return inner

        lax.fori_loop(1 if first else 0, SUP // 2, _pair, 0)
        _process(kb + SUP - 1, SUP - 1, None, *bufs[0])

    _super(0, True)

    def _super_rest(su, carry):
        _super(su, False)
        return carry

    lax.fori_loop(1, NSUP, _super_rest, 0)

    pltpu.make_async_copy(rows0, acc_sh.at[idxbi.at[SUP - 1]], sems0).wait()
    pltpu.make_async_copy(rows1, acc_sh.at[idxbi.at[SUP - 2]], sems1).wait()

    plsc.subcore_barrier()

    def _wb(r, carry):
        m = r * NS + s

        @pl.when(m < NZCH)
        def _():
            pltpu.sync_copy(acc_sh.at[pl.ds(m * ZROWS, ZROWS), :],
                            out_hbm.at[c, pl.ds(m * ZROWS, ZROWS), :])

        return carry

    lax.fori_loop(0, (NZCH + NS - 1) // NS, _wb, 0)


def _scatter_sum(y, fp, idxc):
    mesh = plsc.VectorSubcoreMesh(core_axis_name="c", subcore_axis_name="s",
                                  num_cores=NC, num_subcores=NS)
    fn = pl.kernel(
        _sc_body,
        out_type=jax.ShapeDtypeStruct((NC, N_ATOMS, D), jnp.float32),
        mesh=mesh,
        scratch_types=[
            pltpu.VMEM((SUP, CHUNK), jnp.int32),
            pltpu.VMEM((SUP, CHUNK), jnp.int32),
            pltpu.VMEM((CHUNK, D), jnp.float32),
            pltpu.VMEM((CHUNK, D), jnp.float32),
            pltpu.VMEM((CHUNK, D), jnp.float32),
            pltpu.VMEM((CHUNK, D), jnp.float32),
            pltpu.VMEM_SHARED((N_ATOMS, D), jnp.float32),
            pltpu.SemaphoreType.DMA,
            pltpu.SemaphoreType.DMA,
            pltpu.SemaphoreType.DMA,
            pltpu.SemaphoreType.DMA,
            pltpu.SemaphoreType.DMA,
            pltpu.SemaphoreType.DMA,
        ],
    )
    return fn(y, fp, idxc)


def _post_body(xi_ref, s0_ref, s1_ref,
               w10_ref, b10_ref, w20_ref, b20_ref,
               w11_ref, b11_ref, w21_ref, b21_ref,
               w12_ref, b12_ref, w22_ref, b22_ref,
               wvT_ref, bv_ref, out_ref):
    v = xi_ref[...] + s0_ref[...] + s1_ref[...]
    for w1, b1, w2, b2 in (
        (w10_ref, b10_ref, w20_ref, b20_ref),
        (w11_ref, b11_ref, w21_ref, b21_ref),
        (w12_ref, b12_ref, w22_ref, b22_ref),
    ):
        h = _sp(jnp.dot(v, w1[...], preferred_element_type=jnp.float32) + b1[...])
        v = v + jnp.dot(h, w2[...], preferred_element_type=jnp.float32) + b2[...]
    out_ref[...] = jnp.dot(_sp(v), wvT_ref[...],
                           preferred_element_type=jnp.float32) + bv_ref[...]


def _post(xi, s0, s1, mats, vecs, wvT, bv2):
    blk = 1000
    grid = N_ATOMS // blk
    full = pl.BlockSpec((D, D), lambda i: (0, 0))
    row = pl.BlockSpec((1, D), lambda i: (0, 0))
    big = pl.BlockSpec((blk, D), lambda i: (i, 0))
    in_specs = [big, big, big]
    args = [xi, s0, s1]
    for m, v in zip(mats, vecs):
        in_specs += [full, row]
        args += [m, v]
    in_specs += [full, row]
    args += [wvT, bv2]
    return pl.pallas_call(
        _post_body,
        grid=(grid,),
        in_specs=in_specs,
        out_specs=big,
        out_shape=jax.ShapeDtypeStruct((N_ATOMS, D), jnp.float32),
    )(*args)


def kernel(pair_indices, f_ij, d_ij, atomic_embedding, G, Wi, bi, Wj, bj, Wv, bv,
           res0_W1, res0_b1, res0_W2, res0_b2,
           res1_W1, res1_b1, res1_W2, res1_b2,
           res2_W1, res2_b1, res2_W2, res2_b2):
    idxc = pair_indices.astype(jnp.int32).reshape(2, NW, NSUP, SUP, CHUNK)

    y = _embed_linear(atomic_embedding, Wj.T, bj.reshape(1, D))
    fp = _fprime(f_ij.T, G.T)
    acc = _scatter_sum(y, fp, idxc)
    xi = _embed_linear(atomic_embedding, Wi.T, bi.reshape(1, D))
    mats = (res0_W1.T, res0_W2.T, res1_W1.T, res1_W2.T, res2_W1.T, res2_W2.T)
    vecs = (res0_b1.reshape(1, D), res0_b2.reshape(1, D),
            res1_b1.reshape(1, D), res1_b2.reshape(1, D),
            res2_b1.reshape(1, D), res2_b2.reshape(1, D))
    return _post(xi, acc[0], acc[1], mats, vecs, Wv.T, bv.reshape(1, D))
